# baseline (speedup 1.0000x reference)
import functools

import jax
import jax.numpy as jnp
from jax import lax
from jax.experimental import pallas as pl
from jax.experimental.pallas import tpu as pltpu
from jax.experimental.pallas import tpu_sc as plsc

VOCAB = 100000
EMB = 128
BATCH = 16384

NC = 2
NS = 16
NW = NC * NS
BPW = BATCH // NW
CHUNK = 128
NCHUNK = BPW // CHUNK

_sc_mesh = plsc.VectorSubcoreMesh(core_axis_name="c", subcore_axis_name="s")


@functools.partial(
    pl.kernel,
    out_type=jax.ShapeDtypeStruct((NW, 16), jnp.float32),
    mesh=_sc_mesh,
    compiler_params=pltpu.CompilerParams(needs_layout_passes=False),
    scratch_types=[
        pltpu.VMEM((BPW,), jnp.int32),
        pltpu.VMEM((BPW,), jnp.int32),
        pltpu.VMEM((CHUNK, EMB), jnp.float32),
        pltpu.VMEM((CHUNK, EMB), jnp.float32),
        pltpu.VMEM((CHUNK, EMB), jnp.float32),
        pltpu.VMEM((CHUNK, EMB), jnp.float32),
        pltpu.VMEM((CHUNK, EMB), jnp.float32),
        pltpu.VMEM((CHUNK, EMB), jnp.float32),
        pltpu.VMEM((NCHUNK, CHUNK), jnp.float32),
        pltpu.VMEM((16,), jnp.float32),
        pltpu.SemaphoreType.DMA,
        pltpu.SemaphoreType.DMA,
        pltpu.SemaphoreType.DMA,
    ],
)
def _sc_dots(wi_hbm, wc_hbm, ia_hbm, ic_hbm, out_hbm,
             ia_v, ic_v, a0_v, a1_v, a2_v, c0_v, c1_v, c2_v, dots_v,
             part_v, sem0, sem1, sem2):
    wid = lax.axis_index("s") * NC + lax.axis_index("c")
    cp_ia = pltpu.async_copy(ia_hbm.at[pl.ds(wid * BPW, BPW)], ia_v, sem0)
    cp_ic = pltpu.async_copy(ic_hbm.at[pl.ds(wid * BPW, BPW)], ic_v, sem1)
    cp_ia.wait()
    cp_ic.wait()

    zeros16 = jnp.zeros((16,), jnp.float32)
    for i in range(NCHUNK):
        for b in range(CHUNK // 16):
            dots_v[i, pl.ds(b * 16, 16)] = zeros16

    a_bufs = (a0_v, a1_v, a2_v)
    c_bufs = (c0_v, c1_v, c2_v)
    sems = (sem0, sem1, sem2)
    lane = lax.iota(jnp.int32, 16)
    perm8 = lane ^ 8
    perm4 = lane ^ 4
    perm2 = lane ^ 2
    perm1 = lane ^ 1
    fold_mask = lane < 4

    def fire(c):
        slot = c % 3
        idx_a = ia_v.at[pl.ds(c * CHUNK, CHUNK)]
        idx_c = ic_v.at[pl.ds(c * CHUNK, CHUNK)]
        cp_a = pltpu.async_copy(wi_hbm.at[idx_a], a_bufs[slot], sems[slot])
        cp_c = pltpu.async_copy(wc_hbm.at[idx_c], c_bufs[slot], sems[slot])
        return cp_a, cp_c

    pending = [fire(0), fire(1)]
    acc = jnp.zeros((16,), jnp.float32)
    for c in range(NCHUNK):
        slot = c % 3
        cp_a, cp_c = pending.pop(0)
        cp_a.wait()
        cp_c.wait()
        if c + 2 < NCHUNK:
            pending.append(fire(c + 2))
        a_v, c_v = a_bufs[slot], c_bufs[slot]

        cvec = jnp.full((16,), c, jnp.int32)

        @plsc.parallel_loop(0, CHUNK // 16, 1, unroll=2)
        def _groups(g):
            for j in range(16):
                r = g * 16 + j
                acc0 = (a_v[r, pl.ds(0, 16)] * c_v[r, pl.ds(0, 16)])
                acc1 = (a_v[r, pl.ds(16, 16)] * c_v[r, pl.ds(16, 16)])
                for k in range(2, EMB // 16, 2):
                    acc0 = acc0 + (a_v[r, pl.ds(k * 16, 16)]
                                   * c_v[r, pl.ds(k * 16, 16)])
                    acc1 = acc1 + (a_v[r, pl.ds((k + 1) * 16, 16)]
                                   * c_v[r, pl.ds((k + 1) * 16, 16)])
                p = acc0 + acc1
                p = p + p.at[perm8].get(mode="promise_in_bounds")
                p = p + p.at[perm4].get(mode="promise_in_bounds")
                rvec = jnp.full((16,), r, jnp.int32)
                plsc.addupdate_scatter(dots_v, [cvec, rvec], p,
                                       mask=fold_mask)

        for b in range(CHUNK // 16):
            x = dots_v[c, pl.ds(b * 16, 16)]
            t = jnp.exp(-jnp.abs(x))
            u = t / (2.0 + t)
            u2 = u * u
            poly = 1.0 + u2 * (1.0 / 3.0 + u2 * (1.0 / 5.0 + u2 * (
                1.0 / 7.0 + u2 * (1.0 / 9.0 + u2 * (1.0 / 11.0)))))
            acc = acc + (jnp.minimum(x, 0.0) - 2.0 * u * poly)

    acc = acc + acc.at[perm8].get(mode="promise_in_bounds")
    acc = acc + acc.at[perm4].get(mode="promise_in_bounds")
    acc = acc + acc.at[perm2].get(mode="promise_in_bounds")
    acc = acc + acc.at[perm1].get(mode="promise_in_bounds")
    part_v[...] = acc
    pltpu.sync_copy(part_v, out_hbm.at[wid])


def _loss_body(x_ref, o_ref):
    o_ref[0, 0] = -jnp.sum(x_ref[...][:, 0]) * (1.0 / BATCH)


_loss_call = pl.pallas_call(
    _loss_body,
    out_shape=jax.ShapeDtypeStruct((1, 1), jnp.float32),
    out_specs=pl.BlockSpec(memory_space=pltpu.SMEM),
)


@jax.jit
def kernel(input_word, context_word, W_input, W_context):
    ia = input_word.astype(jnp.int32)
    ic = context_word.astype(jnp.int32)
    parts = _sc_dots(W_input, W_context, ia, ic)
    loss = _loss_call(parts)
    return loss[0, 0]


# speedup vs baseline: 1.0341x; 1.0341x over previous
import functools

import jax
import jax.numpy as jnp
from jax import lax
from jax.experimental import pallas as pl
from jax.experimental.pallas import tpu as pltpu
from jax.experimental.pallas import tpu_sc as plsc

VOCAB = 100000
EMB = 128
BATCH = 16384

NC = 2
NS = 16
NW = NC * NS
BPW = BATCH // NW
CHUNK = 128
NCHUNK = BPW // CHUNK

_sc_mesh = plsc.VectorSubcoreMesh(core_axis_name="c", subcore_axis_name="s")


@functools.partial(
    pl.kernel,
    out_type=jax.ShapeDtypeStruct((BATCH // CHUNK, CHUNK), jnp.float32),
    mesh=_sc_mesh,
    compiler_params=pltpu.CompilerParams(needs_layout_passes=False),
    scratch_types=[
        pltpu.VMEM((BPW,), jnp.int32),
        pltpu.VMEM((BPW,), jnp.int32),
        pltpu.VMEM((CHUNK, EMB), jnp.float32),
        pltpu.VMEM((CHUNK, EMB), jnp.float32),
        pltpu.VMEM((CHUNK, EMB), jnp.float32),
        pltpu.VMEM((CHUNK, EMB), jnp.float32),
        pltpu.VMEM((CHUNK, EMB), jnp.float32),
        pltpu.VMEM((CHUNK, EMB), jnp.float32),
        pltpu.VMEM((NCHUNK, CHUNK), jnp.float32),
        pltpu.SemaphoreType.DMA,
        pltpu.SemaphoreType.DMA,
        pltpu.SemaphoreType.DMA,
    ],
)
def _sc_dots(wi_hbm, wc_hbm, ia_hbm, ic_hbm, out_hbm,
             ia_v, ic_v, a0_v, a1_v, a2_v, c0_v, c1_v, c2_v, dots_v,
             sem0, sem1, sem2):
    wid = lax.axis_index("s") * NC + lax.axis_index("c")
    cp_ia = pltpu.async_copy(ia_hbm.at[pl.ds(wid * BPW, BPW)], ia_v, sem0)
    cp_ic = pltpu.async_copy(ic_hbm.at[pl.ds(wid * BPW, BPW)], ic_v, sem1)
    cp_ia.wait()
    cp_ic.wait()

    zeros16 = jnp.zeros((16,), jnp.float32)
    for i in range(NCHUNK):
        for b in range(CHUNK // 16):
            dots_v[i, pl.ds(b * 16, 16)] = zeros16

    a_bufs = (a0_v, a1_v, a2_v)
    c_bufs = (c0_v, c1_v, c2_v)
    sems = (sem0, sem1, sem2)
    lane = lax.iota(jnp.int32, 16)
    perm8 = lane ^ 8
    perm4 = lane ^ 4
    fold_mask = lane < 4

    def fire(c):
        slot = c % 3
        idx_a = ia_v.at[pl.ds(c * CHUNK, CHUNK)]
        idx_c = ic_v.at[pl.ds(c * CHUNK, CHUNK)]
        cp_a = pltpu.async_copy(wi_hbm.at[idx_a], a_bufs[slot], sems[slot])
        cp_c = pltpu.async_copy(wc_hbm.at[idx_c], c_bufs[slot], sems[slot])
        return cp_a, cp_c

    pending = [fire(0), fire(1)]
    for c in range(NCHUNK):
        slot = c % 3
        cp_a, cp_c = pending.pop(0)
        cp_a.wait()
        cp_c.wait()
        if c + 2 < NCHUNK:
            pending.append(fire(c + 2))
        a_v, c_v = a_bufs[slot], c_bufs[slot]

        cvec = jnp.full((16,), c, jnp.int32)

        @plsc.parallel_loop(0, CHUNK // 16, 1, unroll=2)
        def _groups(g):
            for j in range(16):
                r = g * 16 + j
                acc0 = (a_v[r, pl.ds(0, 16)] * c_v[r, pl.ds(0, 16)])
                acc1 = (a_v[r, pl.ds(16, 16)] * c_v[r, pl.ds(16, 16)])
                for k in range(2, EMB // 16, 2):
                    acc0 = acc0 + (a_v[r, pl.ds(k * 16, 16)]
                                   * c_v[r, pl.ds(k * 16, 16)])
                    acc1 = acc1 + (a_v[r, pl.ds((k + 1) * 16, 16)]
                                   * c_v[r, pl.ds((k + 1) * 16, 16)])
                p = acc0 + acc1
                p = p + p.at[perm8].get(mode="promise_in_bounds")
                p = p + p.at[perm4].get(mode="promise_in_bounds")
                rvec = jnp.full((16,), r, jnp.int32)
                plsc.addupdate_scatter(dots_v, [cvec, rvec], p,
                                       mask=fold_mask)

    pltpu.sync_copy(dots_v, out_hbm.at[pl.ds(wid * NCHUNK, NCHUNK)])


def _loss_body(x_ref, o_ref):
    x = x_ref[...]
    ls = jnp.minimum(x, 0.0) - jnp.log1p(jnp.exp(-jnp.abs(x)))
    o_ref[0, 0] = -jnp.sum(ls) * (1.0 / BATCH)


_loss_call = pl.pallas_call(
    _loss_body,
    out_shape=jax.ShapeDtypeStruct((1, 1), jnp.float32),
    out_specs=pl.BlockSpec(memory_space=pltpu.SMEM),
)


@jax.jit
def kernel(input_word, context_word, W_input, W_context):
    ia = input_word.astype(jnp.int32)
    ic = context_word.astype(jnp.int32)
    dots = _sc_dots(W_input, W_context, ia, ic)
    loss = _loss_call(dots)
    return loss[0, 0]
